# initial kernel scaffold (unmeasured)
import jax
import jax.numpy as jnp
from jax import lax
from jax.experimental import pallas as pl
from jax.experimental.pallas import tpu as pltpu

N_DEV = 4


def kernel(x, w_mat):
    x = x.astype(jnp.bfloat16)
    w = w_mat.astype(jnp.bfloat16)
    m_per, k = x.shape
    n = w.shape[1]

    def body(x_ref, w_ref, out_ref, comm_ref, send_sems, recv_sems):
        my = lax.axis_index("i")
        left = lax.rem(my - 1 + N_DEV, N_DEV)
        right = lax.rem(my + 1, N_DEV)

        barrier_sem = pltpu.get_barrier_semaphore()
        for nbr in (left, right):
            pl.semaphore_signal(
                barrier_sem, inc=1,
                device_id=(nbr,), device_id_type=pl.DeviceIdType.MESH,
            )
        pl.semaphore_wait(barrier_sem, 2)

        def silu(v):
            return v * (1.0 / (1.0 + jnp.exp(-v)))

        comm_ref[0, :, :] = x_ref[:, :]
        y = jnp.dot(x_ref[:, :], w_ref[:, :], preferred_element_type=jnp.float32)
        out_ref[pl.ds(my * m_per, m_per), :] = silu(y)

        for h in range(N_DEV - 1):
            rdma = pltpu.make_async_remote_copy(
                src_ref=comm_ref.at[h],
                dst_ref=comm_ref.at[h + 1],
                send_sem=send_sems.at[h],
                recv_sem=recv_sems.at[h],
                device_id=(right,),
                device_id_type=pl.DeviceIdType.MESH,
            )
            rdma.start()
            rdma.wait()

            origin = lax.rem(my - h - 1 + N_DEV, N_DEV)
            y = jnp.dot(
                comm_ref[h + 1, :, :], w_ref[:, :],
                preferred_element_type=jnp.float32,
            )
            out_ref[pl.ds(origin * m_per, m_per), :] = silu(y)

    return pl.pallas_call(
        body,
        out_shape=jax.ShapeDtypeStruct((N_DEV * m_per, n), jnp.float32),
        in_specs=[
            pl.BlockSpec(memory_space=pltpu.VMEM),
            pl.BlockSpec(memory_space=pltpu.VMEM),
        ],
        out_specs=pl.BlockSpec(memory_space=pltpu.VMEM),
        scratch_shapes=[
            pltpu.VMEM((N_DEV, m_per, k), jnp.bfloat16),
            pltpu.SemaphoreType.DMA((N_DEV - 1,)),
            pltpu.SemaphoreType.DMA((N_DEV - 1,)),
        ],
        compiler_params=pltpu.CompilerParams(collective_id=0),
    )(x, w)


# baseline (device time: 327110 ns/iter reference)
import jax
import jax.numpy as jnp
from jax import lax
from jax.experimental import pallas as pl
from jax.experimental.pallas import tpu as pltpu

N_DEV = 4


def kernel(x, w_mat):
    x = x.astype(jnp.bfloat16)
    w = w_mat.astype(jnp.bfloat16)
    m_per, k = x.shape
    n = w.shape[1]

    def body(x_ref, w_ref, out_ref, comm_ref, send_sems, recv_sems):
        my = lax.axis_index("i")
        left = lax.rem(my - 1 + N_DEV, N_DEV)
        right = lax.rem(my + 1, N_DEV)

        barrier_sem = pltpu.get_barrier_semaphore()
        for nbr in (left, right):
            pl.semaphore_signal(
                barrier_sem, inc=1,
                device_id=(nbr,), device_id_type=pl.DeviceIdType.MESH,
            )
        pl.semaphore_wait(barrier_sem, 2)

        def silu(v):
            return v * (1.0 / (1.0 + jnp.exp(-v)))

        comm_ref[0, :, :] = x_ref[:, :]
        y = jnp.dot(x_ref[:, :], w_ref[:, :], preferred_element_type=jnp.float32)
        out_ref[pl.ds(my * m_per, m_per), :] = silu(y)

        for h in range(N_DEV - 1):
            rdma = pltpu.make_async_remote_copy(
                src_ref=comm_ref.at[h],
                dst_ref=comm_ref.at[h + 1],
                send_sem=send_sems.at[h],
                recv_sem=recv_sems.at[h],
                device_id=(right,),
                device_id_type=pl.DeviceIdType.MESH,
            )
            rdma.start()
            rdma.wait()

            origin = lax.rem(my - h - 1 + N_DEV, N_DEV)
            y = jnp.dot(
                comm_ref[h + 1, :, :], w_ref[:, :],
                preferred_element_type=jnp.float32,
            )
            out_ref[pl.ds(origin * m_per, m_per), :] = silu(y)

    return pl.pallas_call(
        body,
        out_shape=jax.ShapeDtypeStruct((N_DEV * m_per, n), jnp.float32),
        in_specs=[
            pl.BlockSpec(memory_space=pltpu.VMEM),
            pl.BlockSpec(memory_space=pltpu.VMEM),
        ],
        out_specs=pl.BlockSpec(memory_space=pltpu.VMEM),
        scratch_shapes=[
            pltpu.VMEM((N_DEV, m_per, k), jnp.bfloat16),
            pltpu.SemaphoreType.DMA((N_DEV - 1,)),
            pltpu.SemaphoreType.DMA((N_DEV - 1,)),
        ],
        compiler_params=pltpu.CompilerParams(
            collective_id=0,
            vmem_limit_bytes=100 * 1024 * 1024,
        ),
    )(x, w)


# device time: 177506 ns/iter; 1.8428x vs baseline; 1.8428x over previous
import jax
import jax.numpy as jnp
from jax import lax
from jax.experimental import pallas as pl
from jax.experimental.pallas import tpu as pltpu

N_DEV = 4
N_HOP = N_DEV - 1


def kernel(x, w_mat):
    x = x.astype(jnp.bfloat16)
    w = w_mat.astype(jnp.bfloat16)
    m_per, k = x.shape
    n = w.shape[1]
    half = m_per // 2

    def body(x_ref, w_ref, out_ref, comm_top, comm_bot,
             send_t, recv_t, send_b, recv_b):
        my = lax.axis_index("i")
        left = (my + N_DEV - 1) % N_DEV
        right = (my + 1) % N_DEV

        barrier_sem = pltpu.get_barrier_semaphore()
        for nbr in (left, right):
            pl.semaphore_signal(
                barrier_sem, inc=1,
                device_id=(nbr,), device_id_type=pl.DeviceIdType.MESH,
            )
        pl.semaphore_wait(barrier_sem, 2)

        def silu(v):
            return v * (1.0 / (1.0 + jnp.exp(-v)))

        def mm(a):
            return jnp.dot(a, w_ref[:, :], preferred_element_type=jnp.float32)

        rdma_t = pltpu.make_async_remote_copy(
            src_ref=x_ref.at[pl.ds(0, half), :],
            dst_ref=comm_top.at[0],
            send_sem=send_t.at[0], recv_sem=recv_t.at[0],
            device_id=(right,), device_id_type=pl.DeviceIdType.MESH,
        )
        rdma_b = pltpu.make_async_remote_copy(
            src_ref=x_ref.at[pl.ds(half, half), :],
            dst_ref=comm_bot.at[0],
            send_sem=send_b.at[0], recv_sem=recv_b.at[0],
            device_id=(left,), device_id_type=pl.DeviceIdType.MESH,
        )
        rdma_t.start()
        rdma_b.start()
        started = [rdma_t, rdma_b]

        out_ref[pl.ds(my * m_per, m_per), :] = silu(mm(x_ref[:, :]))

        for h in range(N_HOP):
            rdma_t.wait_recv()
            if h + 1 < N_HOP:
                rdma_t = pltpu.make_async_remote_copy(
                    src_ref=comm_top.at[h],
                    dst_ref=comm_top.at[h + 1],
                    send_sem=send_t.at[h + 1], recv_sem=recv_t.at[h + 1],
                    device_id=(right,), device_id_type=pl.DeviceIdType.MESH,
                )
                rdma_t.start()
                started.append(rdma_t)
            rdma_b.wait_recv()
            if h + 1 < N_HOP:
                rdma_b = pltpu.make_async_remote_copy(
                    src_ref=comm_bot.at[h],
                    dst_ref=comm_bot.at[h + 1],
                    send_sem=send_b.at[h + 1], recv_sem=recv_b.at[h + 1],
                    device_id=(left,), device_id_type=pl.DeviceIdType.MESH,
                )
                rdma_b.start()
                started.append(rdma_b)

            o_t = (my + N_DEV - 1 - h) % N_DEV
            o_b = (my + 1 + h) % N_DEV
            out_ref[pl.ds(o_t * m_per, half), :] = silu(mm(comm_top[h, :, :]))
            out_ref[pl.ds(o_b * m_per + half, half), :] = silu(mm(comm_bot[h, :, :]))

        for r in started:
            r.wait_send()

    return pl.pallas_call(
        body,
        out_shape=jax.ShapeDtypeStruct((N_DEV * m_per, n), jnp.float32),
        in_specs=[
            pl.BlockSpec(memory_space=pltpu.VMEM),
            pl.BlockSpec(memory_space=pltpu.VMEM),
        ],
        out_specs=pl.BlockSpec(memory_space=pltpu.VMEM),
        scratch_shapes=[
            pltpu.VMEM((N_HOP, half, k), jnp.bfloat16),
            pltpu.VMEM((N_HOP, half, k), jnp.bfloat16),
            pltpu.SemaphoreType.DMA((N_HOP,)),
            pltpu.SemaphoreType.DMA((N_HOP,)),
            pltpu.SemaphoreType.DMA((N_HOP,)),
            pltpu.SemaphoreType.DMA((N_HOP,)),
        ],
        compiler_params=pltpu.CompilerParams(
            collective_id=0,
            vmem_limit_bytes=100 * 1024 * 1024,
        ),
    )(x, w)


# device time: 163031 ns/iter; 2.0064x vs baseline; 1.0888x over previous
import jax
import jax.numpy as jnp
from jax import lax
from jax.experimental import pallas as pl
from jax.experimental.pallas import tpu as pltpu

N_DEV = 4
N_HOP = N_DEV - 1


def kernel(x, w_mat):
    m_per, k = x.shape
    n = w_mat.shape[1]
    half = m_per // 2
    kh = k // 2

    def body(x_ref, w_hbm, out_hbm, own_bf, w_stage, w_bf, out_stage,
             comm_top, comm_bot, send_t, recv_t, send_b, recv_b,
             out_sems, w_sem):
        my = lax.axis_index("i")
        left = (my + N_DEV - 1) % N_DEV
        right = (my + 1) % N_DEV

        barrier_sem = pltpu.get_barrier_semaphore()
        for nbr in (left, right):
            pl.semaphore_signal(
                barrier_sem, inc=1,
                device_id=(nbr,), device_id_type=pl.DeviceIdType.MESH,
            )
        pl.semaphore_wait(barrier_sem, 2)

        def silu(v):
            return v * (1.0 / (1.0 + jnp.exp(-v)))

        def mm(a):
            return jnp.dot(a, w_bf[:, :], preferred_element_type=jnp.float32)

        own_bf[pl.ds(0, half), :] = x_ref[pl.ds(0, half), :].astype(jnp.bfloat16)
        rdma_t = pltpu.make_async_remote_copy(
            src_ref=own_bf.at[pl.ds(0, half), :],
            dst_ref=comm_top.at[0],
            send_sem=send_t.at[0], recv_sem=recv_t.at[0],
            device_id=(right,), device_id_type=pl.DeviceIdType.MESH,
        )
        rdma_t.start()
        own_bf[pl.ds(half, half), :] = x_ref[pl.ds(half, half), :].astype(jnp.bfloat16)
        rdma_b = pltpu.make_async_remote_copy(
            src_ref=own_bf.at[pl.ds(half, half), :],
            dst_ref=comm_bot.at[0],
            send_sem=send_b.at[0], recv_sem=recv_b.at[0],
            device_id=(left,), device_id_type=pl.DeviceIdType.MESH,
        )
        rdma_b.start()
        started = [rdma_t, rdma_b]

        for i in range(2):
            cp = pltpu.make_async_copy(
                w_hbm.at[pl.ds(i * kh, kh), :], w_stage, w_sem,
            )
            cp.start()
            cp.wait()
            w_bf[pl.ds(i * kh, kh), :] = w_stage[:, :].astype(jnp.bfloat16)

        pending = [None, None]

        def emit(tile, row_start, slot):
            if pending[slot] is not None:
                pending[slot].wait()
            out_stage[slot, :, :] = tile
            cp = pltpu.make_async_copy(
                out_stage.at[slot],
                out_hbm.at[pl.ds(row_start, half), :],
                out_sems.at[slot],
            )
            cp.start()
            pending[slot] = cp

        emit(silu(mm(own_bf[pl.ds(0, half), :])), my * m_per, 0)
        emit(silu(mm(own_bf[pl.ds(half, half), :])), my * m_per + half, 1)

        for h in range(N_HOP):
            rdma_t.wait_recv()
            if h + 1 < N_HOP:
                rdma_t = pltpu.make_async_remote_copy(
                    src_ref=comm_top.at[h],
                    dst_ref=comm_top.at[h + 1],
                    send_sem=send_t.at[h + 1], recv_sem=recv_t.at[h + 1],
                    device_id=(right,), device_id_type=pl.DeviceIdType.MESH,
                )
                rdma_t.start()
                started.append(rdma_t)
            rdma_b.wait_recv()
            if h + 1 < N_HOP:
                rdma_b = pltpu.make_async_remote_copy(
                    src_ref=comm_bot.at[h],
                    dst_ref=comm_bot.at[h + 1],
                    send_sem=send_b.at[h + 1], recv_sem=recv_b.at[h + 1],
                    device_id=(left,), device_id_type=pl.DeviceIdType.MESH,
                )
                rdma_b.start()
                started.append(rdma_b)

            o_t = (my + N_DEV - 1 - h) % N_DEV
            o_b = (my + 1 + h) % N_DEV
            emit(silu(mm(comm_top[h, :, :])), o_t * m_per, 0)
            emit(silu(mm(comm_bot[h, :, :])), o_b * m_per + half, 1)

        for r in started:
            r.wait_send()
        for p in pending:
            p.wait()

    return pl.pallas_call(
        body,
        out_shape=jax.ShapeDtypeStruct((N_DEV * m_per, n), jnp.float32),
        in_specs=[
            pl.BlockSpec(memory_space=pltpu.VMEM),
            pl.BlockSpec(memory_space=pl.ANY),
        ],
        out_specs=pl.BlockSpec(memory_space=pl.ANY),
        scratch_shapes=[
            pltpu.VMEM((m_per, k), jnp.bfloat16),
            pltpu.VMEM((kh, n), jnp.float32),
            pltpu.VMEM((k, n), jnp.bfloat16),
            pltpu.VMEM((2, half, n), jnp.float32),
            pltpu.VMEM((N_HOP, half, k), jnp.bfloat16),
            pltpu.VMEM((N_HOP, half, k), jnp.bfloat16),
            pltpu.SemaphoreType.DMA((N_HOP,)),
            pltpu.SemaphoreType.DMA((N_HOP,)),
            pltpu.SemaphoreType.DMA((N_HOP,)),
            pltpu.SemaphoreType.DMA((N_HOP,)),
            pltpu.SemaphoreType.DMA((2,)),
            pltpu.SemaphoreType.DMA,
        ],
        compiler_params=pltpu.CompilerParams(
            collective_id=0,
            vmem_limit_bytes=100 * 1024 * 1024,
        ),
    )(x, w_mat)


# device time: 152885 ns/iter; 2.1396x vs baseline; 1.0664x over previous
import jax
import jax.numpy as jnp
from jax import lax
from jax.experimental import pallas as pl
from jax.experimental.pallas import tpu as pltpu

N_DEV = 4
N_HOP = N_DEV - 1
N_SUB = 2


def kernel(x, w_mat):
    m_per, k = x.shape
    n = w_mat.shape[1]
    half = m_per // 2
    sub = half // N_SUB
    kh = k // 2

    def body(x_hbm, w_hbm, out_hbm, x_stage, own_bf, w_stage, w_bf,
             out_stage, comm_top, comm_bot, send_t, recv_t, send_b, recv_b,
             x_sems, out_sems, w_sem):
        my = lax.axis_index("i")
        left = (my + N_DEV - 1) % N_DEV
        right = (my + 1) % N_DEV

        q_order = (0, 2, 1, 3)
        x_cp = {}
        for j, q in enumerate(q_order[:2]):
            x_cp[q] = pltpu.make_async_copy(
                x_hbm.at[pl.ds(q * sub, sub), :], x_stage.at[j % 2],
                x_sems.at[j % 2],
            )
            x_cp[q].start()

        barrier_sem = pltpu.get_barrier_semaphore()
        for nbr in (left, right):
            pl.semaphore_signal(
                barrier_sem, inc=1,
                device_id=(nbr,), device_id_type=pl.DeviceIdType.MESH,
            )
        pl.semaphore_wait(barrier_sem, 2)

        def silu(v):
            return v * (1.0 / (1.0 + jnp.exp(-v)))

        def mm(a):
            return jnp.dot(a, w_bf[:, :], preferred_element_type=jnp.float32)

        started = []
        for j, q in enumerate(q_order):
            slot = j % 2
            x_cp[q].wait()
            own_bf[q, :, :] = x_stage[slot, :, :].astype(jnp.bfloat16)
            if j + 2 < 4:
                nq = q_order[j + 2]
                x_cp[nq] = pltpu.make_async_copy(
                    x_hbm.at[pl.ds(nq * sub, sub), :], x_stage.at[slot],
                    x_sems.at[slot],
                )
                x_cp[nq].start()
            if q < 2:
                rdma = pltpu.make_async_remote_copy(
                    src_ref=own_bf.at[q],
                    dst_ref=comm_top.at[0, q],
                    send_sem=send_t.at[0, q], recv_sem=recv_t.at[0, q],
                    device_id=(right,), device_id_type=pl.DeviceIdType.MESH,
                )
            else:
                rdma = pltpu.make_async_remote_copy(
                    src_ref=own_bf.at[q],
                    dst_ref=comm_bot.at[0, q - 2],
                    send_sem=send_b.at[0, q - 2], recv_sem=recv_b.at[0, q - 2],
                    device_id=(left,), device_id_type=pl.DeviceIdType.MESH,
                )
            rdma.start()
            started.append(rdma)

        for i in range(2):
            cp = pltpu.make_async_copy(
                w_hbm.at[pl.ds(i * kh, kh), :], w_stage, w_sem,
            )
            cp.start()
            cp.wait()
            w_bf[pl.ds(i * kh, kh), :] = w_stage[:, :].astype(jnp.bfloat16)

        pending = [None, None]
        emit_n = [0]

        def emit(tile, row_start):
            slot = emit_n[0] % 2
            emit_n[0] += 1
            if pending[slot] is not None:
                pending[slot].wait()
            out_stage[slot, :, :] = tile
            cp = pltpu.make_async_copy(
                out_stage.at[slot],
                out_hbm.at[pl.ds(row_start, sub), :],
                out_sems.at[slot],
            )
            cp.start()
            pending[slot] = cp

        for q in range(4):
            emit(silu(mm(own_bf[q, :, :])), my * m_per + q * sub)

        for h in range(N_HOP):
            o_t = (my + N_DEV - 1 - h) % N_DEV
            o_b = (my + 1 + h) % N_DEV
            for s in range(N_SUB):
                recv_wait_t = pltpu.make_async_remote_copy(
                    src_ref=comm_top.at[h, s], dst_ref=comm_top.at[h, s],
                    send_sem=send_t.at[h, s], recv_sem=recv_t.at[h, s],
                    device_id=(right,), device_id_type=pl.DeviceIdType.MESH,
                )
                recv_wait_t.wait_recv()
                if h + 1 < N_HOP:
                    fwd = pltpu.make_async_remote_copy(
                        src_ref=comm_top.at[h, s],
                        dst_ref=comm_top.at[h + 1, s],
                        send_sem=send_t.at[h + 1, s],
                        recv_sem=recv_t.at[h + 1, s],
                        device_id=(right,), device_id_type=pl.DeviceIdType.MESH,
                    )
                    fwd.start()
                    started.append(fwd)
                recv_wait_b = pltpu.make_async_remote_copy(
                    src_ref=comm_bot.at[h, s], dst_ref=comm_bot.at[h, s],
                    send_sem=send_b.at[h, s], recv_sem=recv_b.at[h, s],
                    device_id=(left,), device_id_type=pl.DeviceIdType.MESH,
                )
                recv_wait_b.wait_recv()
                if h + 1 < N_HOP:
                    fwd = pltpu.make_async_remote_copy(
                        src_ref=comm_bot.at[h, s],
                        dst_ref=comm_bot.at[h + 1, s],
                        send_sem=send_b.at[h + 1, s],
                        recv_sem=recv_b.at[h + 1, s],
                        device_id=(left,), device_id_type=pl.DeviceIdType.MESH,
                    )
                    fwd.start()
                    started.append(fwd)

                emit(silu(mm(comm_top[h, s, :, :])), o_t * m_per + s * sub)
                emit(silu(mm(comm_bot[h, s, :, :])),
                     o_b * m_per + half + s * sub)

        for r in started:
            r.wait_send()
        for p in pending:
            p.wait()

    return pl.pallas_call(
        body,
        out_shape=jax.ShapeDtypeStruct((N_DEV * m_per, n), jnp.float32),
        in_specs=[
            pl.BlockSpec(memory_space=pl.ANY),
            pl.BlockSpec(memory_space=pl.ANY),
        ],
        out_specs=pl.BlockSpec(memory_space=pl.ANY),
        scratch_shapes=[
            pltpu.VMEM((2, sub, k), jnp.float32),
            pltpu.VMEM((4, sub, k), jnp.bfloat16),
            pltpu.VMEM((kh, n), jnp.float32),
            pltpu.VMEM((k, n), jnp.bfloat16),
            pltpu.VMEM((2, sub, n), jnp.float32),
            pltpu.VMEM((N_HOP, N_SUB, sub, k), jnp.bfloat16),
            pltpu.VMEM((N_HOP, N_SUB, sub, k), jnp.bfloat16),
            pltpu.SemaphoreType.DMA((N_HOP, N_SUB)),
            pltpu.SemaphoreType.DMA((N_HOP, N_SUB)),
            pltpu.SemaphoreType.DMA((N_HOP, N_SUB)),
            pltpu.SemaphoreType.DMA((N_HOP, N_SUB)),
            pltpu.SemaphoreType.DMA((2,)),
            pltpu.SemaphoreType.DMA((2,)),
            pltpu.SemaphoreType.DMA,
        ],
        compiler_params=pltpu.CompilerParams(
            collective_id=0,
            vmem_limit_bytes=100 * 1024 * 1024,
        ),
    )(x, w_mat)


# device time: 150400 ns/iter; 2.1749x vs baseline; 1.0165x over previous
import jax
import jax.numpy as jnp
from jax import lax
from jax.experimental import pallas as pl
from jax.experimental.pallas import tpu as pltpu

N_DEV = 4
N_HOP = N_DEV - 1
N_SUB = 4
N_PIECE = 2 * N_SUB


def kernel(x, w_mat):
    m_per, k = x.shape
    n = w_mat.shape[1]
    half = m_per // 2
    sub = half // N_SUB
    kh = k // 2

    def body(x_hbm, w_hbm, out_hbm, x_stage, own_bf, w_stage, w_bf,
             out_stage, comm_top, comm_bot, send_t, recv_t, send_b, recv_b,
             x_sems, out_sems, w_sem):
        my = lax.axis_index("i")
        left = (my + N_DEV - 1) % N_DEV
        right = (my + 1) % N_DEV

        q_order = tuple(
            p for s in range(N_SUB) for p in (s, N_SUB + s)
        )
        x_cp = {}
        for j, q in enumerate(q_order[:2]):
            x_cp[q] = pltpu.make_async_copy(
                x_hbm.at[pl.ds(q * sub, sub), :], x_stage.at[j % 2],
                x_sems.at[j % 2],
            )
            x_cp[q].start()

        barrier_sem = pltpu.get_barrier_semaphore()
        for nbr in (left, right):
            pl.semaphore_signal(
                barrier_sem, inc=1,
                device_id=(nbr,), device_id_type=pl.DeviceIdType.MESH,
            )
        pl.semaphore_wait(barrier_sem, 2)

        def silu(v):
            return v * (1.0 / (1.0 + jnp.exp(-v)))

        def mm(a):
            return jnp.dot(a, w_bf[:, :], preferred_element_type=jnp.float32)

        started = []
        for j, q in enumerate(q_order):
            slot = j % 2
            x_cp[q].wait()
            own_bf[q, :, :] = x_stage[slot, :, :].astype(jnp.bfloat16)
            if j + 2 < N_PIECE:
                nq = q_order[j + 2]
                x_cp[nq] = pltpu.make_async_copy(
                    x_hbm.at[pl.ds(nq * sub, sub), :], x_stage.at[slot],
                    x_sems.at[slot],
                )
                x_cp[nq].start()
            if q < N_SUB:
                rdma = pltpu.make_async_remote_copy(
                    src_ref=own_bf.at[q],
                    dst_ref=comm_top.at[0, q],
                    send_sem=send_t.at[0, q], recv_sem=recv_t.at[0, q],
                    device_id=(right,), device_id_type=pl.DeviceIdType.MESH,
                )
            else:
                rdma = pltpu.make_async_remote_copy(
                    src_ref=own_bf.at[q],
                    dst_ref=comm_bot.at[0, q - N_SUB],
                    send_sem=send_b.at[0, q - N_SUB],
                    recv_sem=recv_b.at[0, q - N_SUB],
                    device_id=(left,), device_id_type=pl.DeviceIdType.MESH,
                )
            rdma.start()
            started.append(rdma)

        for i in range(2):
            cp = pltpu.make_async_copy(
                w_hbm.at[pl.ds(i * kh, kh), :], w_stage, w_sem,
            )
            cp.start()
            cp.wait()
            w_bf[pl.ds(i * kh, kh), :] = w_stage[:, :].astype(jnp.bfloat16)

        pending = [None, None]
        emit_n = [0]

        def emit(tile, row_start):
            slot = emit_n[0] % 2
            emit_n[0] += 1
            if pending[slot] is not None:
                pending[slot].wait()
            out_stage[slot, :, :] = tile
            cp = pltpu.make_async_copy(
                out_stage.at[slot],
                out_hbm.at[pl.ds(row_start, sub), :],
                out_sems.at[slot],
            )
            cp.start()
            pending[slot] = cp

        for q in range(N_PIECE):
            emit(silu(mm(own_bf[q, :, :])), my * m_per + q * sub)

        for h in range(N_HOP):
            o_t = (my + N_DEV - 1 - h) % N_DEV
            o_b = (my + 1 + h) % N_DEV
            for s in range(N_SUB):
                recv_wait_t = pltpu.make_async_remote_copy(
                    src_ref=comm_top.at[h, s], dst_ref=comm_top.at[h, s],
                    send_sem=send_t.at[h, s], recv_sem=recv_t.at[h, s],
                    device_id=(right,), device_id_type=pl.DeviceIdType.MESH,
                )
                recv_wait_t.wait_recv()
                if h + 1 < N_HOP:
                    fwd = pltpu.make_async_remote_copy(
                        src_ref=comm_top.at[h, s],
                        dst_ref=comm_top.at[h + 1, s],
                        send_sem=send_t.at[h + 1, s],
                        recv_sem=recv_t.at[h + 1, s],
                        device_id=(right,), device_id_type=pl.DeviceIdType.MESH,
                    )
                    fwd.start()
                    started.append(fwd)
                recv_wait_b = pltpu.make_async_remote_copy(
                    src_ref=comm_bot.at[h, s], dst_ref=comm_bot.at[h, s],
                    send_sem=send_b.at[h, s], recv_sem=recv_b.at[h, s],
                    device_id=(left,), device_id_type=pl.DeviceIdType.MESH,
                )
                recv_wait_b.wait_recv()
                if h + 1 < N_HOP:
                    fwd = pltpu.make_async_remote_copy(
                        src_ref=comm_bot.at[h, s],
                        dst_ref=comm_bot.at[h + 1, s],
                        send_sem=send_b.at[h + 1, s],
                        recv_sem=recv_b.at[h + 1, s],
                        device_id=(left,), device_id_type=pl.DeviceIdType.MESH,
                    )
                    fwd.start()
                    started.append(fwd)

                emit(silu(mm(comm_top[h, s, :, :])), o_t * m_per + s * sub)
                emit(silu(mm(comm_bot[h, s, :, :])),
                     o_b * m_per + half + s * sub)

        for r in started:
            r.wait_send()
        for p in pending:
            p.wait()

    return pl.pallas_call(
        body,
        out_shape=jax.ShapeDtypeStruct((N_DEV * m_per, n), jnp.float32),
        in_specs=[
            pl.BlockSpec(memory_space=pl.ANY),
            pl.BlockSpec(memory_space=pl.ANY),
        ],
        out_specs=pl.BlockSpec(memory_space=pl.ANY),
        scratch_shapes=[
            pltpu.VMEM((2, sub, k), jnp.float32),
            pltpu.VMEM((N_PIECE, sub, k), jnp.bfloat16),
            pltpu.VMEM((kh, n), jnp.float32),
            pltpu.VMEM((k, n), jnp.bfloat16),
            pltpu.VMEM((2, sub, n), jnp.float32),
            pltpu.VMEM((N_HOP, N_SUB, sub, k), jnp.bfloat16),
            pltpu.VMEM((N_HOP, N_SUB, sub, k), jnp.bfloat16),
            pltpu.SemaphoreType.DMA((N_HOP, N_SUB)),
            pltpu.SemaphoreType.DMA((N_HOP, N_SUB)),
            pltpu.SemaphoreType.DMA((N_HOP, N_SUB)),
            pltpu.SemaphoreType.DMA((N_HOP, N_SUB)),
            pltpu.SemaphoreType.DMA((2,)),
            pltpu.SemaphoreType.DMA((2,)),
            pltpu.SemaphoreType.DMA,
        ],
        compiler_params=pltpu.CompilerParams(
            collective_id=0,
            vmem_limit_bytes=100 * 1024 * 1024,
        ),
    )(x, w_mat)


# device time: 150057 ns/iter; 2.1799x vs baseline; 1.0023x over previous
import jax
import jax.numpy as jnp
from jax import lax
from jax.experimental import pallas as pl
from jax.experimental.pallas import tpu as pltpu

N_DEV = 4
N_HOP = N_DEV - 1
N_SUB = 8
N_PIECE = 2 * N_SUB


def kernel(x, w_mat):
    m_per, k = x.shape
    n = w_mat.shape[1]
    half = m_per // 2
    sub = half // N_SUB
    kh = k // 2

    def body(x_hbm, w_hbm, out_hbm, x_stage, own_bf, w_stage, w_bf,
             out_stage, comm_top, comm_bot, send_t, recv_t, send_b, recv_b,
             x_sems, out_sems, w_sem):
        my = lax.axis_index("i")
        left = (my + N_DEV - 1) % N_DEV
        right = (my + 1) % N_DEV

        q_order = tuple(
            p for s in range(N_SUB) for p in (s, N_SUB + s)
        )
        x_cp = {}
        for j, q in enumerate(q_order[:2]):
            x_cp[q] = pltpu.make_async_copy(
                x_hbm.at[pl.ds(q * sub, sub), :], x_stage.at[j % 2],
                x_sems.at[j % 2],
            )
            x_cp[q].start()

        barrier_sem = pltpu.get_barrier_semaphore()
        for nbr in (left, right):
            pl.semaphore_signal(
                barrier_sem, inc=1,
                device_id=(nbr,), device_id_type=pl.DeviceIdType.MESH,
            )
        pl.semaphore_wait(barrier_sem, 2)

        def silu(v):
            return v * (1.0 / (1.0 + jnp.exp(-v)))

        def mm(a):
            return jnp.dot(a, w_bf[:, :], preferred_element_type=jnp.float32)

        started = []
        for j, q in enumerate(q_order):
            slot = j % 2
            x_cp[q].wait()
            own_bf[q, :, :] = x_stage[slot, :, :].astype(jnp.bfloat16)
            if j + 2 < N_PIECE:
                nq = q_order[j + 2]
                x_cp[nq] = pltpu.make_async_copy(
                    x_hbm.at[pl.ds(nq * sub, sub), :], x_stage.at[slot],
                    x_sems.at[slot],
                )
                x_cp[nq].start()
            if q < N_SUB:
                rdma = pltpu.make_async_remote_copy(
                    src_ref=own_bf.at[q],
                    dst_ref=comm_top.at[0, q],
                    send_sem=send_t.at[0, q], recv_sem=recv_t.at[0, q],
                    device_id=(right,), device_id_type=pl.DeviceIdType.MESH,
                )
            else:
                rdma = pltpu.make_async_remote_copy(
                    src_ref=own_bf.at[q],
                    dst_ref=comm_bot.at[0, q - N_SUB],
                    send_sem=send_b.at[0, q - N_SUB],
                    recv_sem=recv_b.at[0, q - N_SUB],
                    device_id=(left,), device_id_type=pl.DeviceIdType.MESH,
                )
            rdma.start()
            started.append(rdma)

        for i in range(2):
            cp = pltpu.make_async_copy(
                w_hbm.at[pl.ds(i * kh, kh), :], w_stage, w_sem,
            )
            cp.start()
            cp.wait()
            w_bf[pl.ds(i * kh, kh), :] = w_stage[:, :].astype(jnp.bfloat16)

        pending = [None, None]
        emit_n = [0]

        def emit(tile, row_start):
            slot = emit_n[0] % 2
            emit_n[0] += 1
            if pending[slot] is not None:
                pending[slot].wait()
            out_stage[slot, :, :] = tile
            cp = pltpu.make_async_copy(
                out_stage.at[slot],
                out_hbm.at[pl.ds(row_start, sub), :],
                out_sems.at[slot],
            )
            cp.start()
            pending[slot] = cp

        for q in range(N_PIECE):
            emit(silu(mm(own_bf[q, :, :])), my * m_per + q * sub)

        for h in range(N_HOP):
            o_t = (my + N_DEV - 1 - h) % N_DEV
            o_b = (my + 1 + h) % N_DEV
            for s in range(N_SUB):
                recv_wait_t = pltpu.make_async_remote_copy(
                    src_ref=comm_top.at[h, s], dst_ref=comm_top.at[h, s],
                    send_sem=send_t.at[h, s], recv_sem=recv_t.at[h, s],
                    device_id=(right,), device_id_type=pl.DeviceIdType.MESH,
                )
                recv_wait_t.wait_recv()
                if h + 1 < N_HOP:
                    fwd = pltpu.make_async_remote_copy(
                        src_ref=comm_top.at[h, s],
                        dst_ref=comm_top.at[h + 1, s],
                        send_sem=send_t.at[h + 1, s],
                        recv_sem=recv_t.at[h + 1, s],
                        device_id=(right,), device_id_type=pl.DeviceIdType.MESH,
                    )
                    fwd.start()
                    started.append(fwd)
                recv_wait_b = pltpu.make_async_remote_copy(
                    src_ref=comm_bot.at[h, s], dst_ref=comm_bot.at[h, s],
                    send_sem=send_b.at[h, s], recv_sem=recv_b.at[h, s],
                    device_id=(left,), device_id_type=pl.DeviceIdType.MESH,
                )
                recv_wait_b.wait_recv()
                if h + 1 < N_HOP:
                    fwd = pltpu.make_async_remote_copy(
                        src_ref=comm_bot.at[h, s],
                        dst_ref=comm_bot.at[h + 1, s],
                        send_sem=send_b.at[h + 1, s],
                        recv_sem=recv_b.at[h + 1, s],
                        device_id=(left,), device_id_type=pl.DeviceIdType.MESH,
                    )
                    fwd.start()
                    started.append(fwd)

                emit(silu(mm(comm_top[h, s, :, :])), o_t * m_per + s * sub)
                emit(silu(mm(comm_bot[h, s, :, :])),
                     o_b * m_per + half + s * sub)

        for r in started:
            r.wait_send()
        for p in pending:
            p.wait()

    return pl.pallas_call(
        body,
        out_shape=jax.ShapeDtypeStruct((N_DEV * m_per, n), jnp.float32),
        in_specs=[
            pl.BlockSpec(memory_space=pl.ANY),
            pl.BlockSpec(memory_space=pl.ANY),
        ],
        out_specs=pl.BlockSpec(memory_space=pl.ANY),
        scratch_shapes=[
            pltpu.VMEM((2, sub, k), jnp.float32),
            pltpu.VMEM((N_PIECE, sub, k), jnp.bfloat16),
            pltpu.VMEM((kh, n), jnp.float32),
            pltpu.VMEM((k, n), jnp.bfloat16),
            pltpu.VMEM((2, sub, n), jnp.float32),
            pltpu.VMEM((N_HOP, N_SUB, sub, k), jnp.bfloat16),
            pltpu.VMEM((N_HOP, N_SUB, sub, k), jnp.bfloat16),
            pltpu.SemaphoreType.DMA((N_HOP, N_SUB)),
            pltpu.SemaphoreType.DMA((N_HOP, N_SUB)),
            pltpu.SemaphoreType.DMA((N_HOP, N_SUB)),
            pltpu.SemaphoreType.DMA((N_HOP, N_SUB)),
            pltpu.SemaphoreType.DMA((2,)),
            pltpu.SemaphoreType.DMA((2,)),
            pltpu.SemaphoreType.DMA,
        ],
        compiler_params=pltpu.CompilerParams(
            collective_id=0,
            vmem_limit_bytes=100 * 1024 * 1024,
        ),
    )(x, w_mat)
